# baseline (speedup 1.0000x reference)
import numpy as np
import jax
import jax.numpy as jnp
from jax.experimental import pallas as pl
from jax.experimental.pallas import tpu as pltpu

B, T, S = 4096, 10000, 128
BR = 256
NBLK = B // BR
EDGE_TILE = T // 128
EDGE_VALID = T - EDGE_TILE * 128


def _subsample_taps():
    max_logspace = np.log10(T - 1)
    idx = np.round(np.logspace(0.0, max_logspace, S, endpoint=True), 1).astype(int)
    idx[0] = 0
    return idx.astype(np.int32)


def _build_plan():
    taps = _subsample_taps()
    tiles = sorted(set(int(t) // 128 for t in taps if t // 128 != EDGE_TILE))
    pos = {c: k for k, c in enumerate(tiles)}
    w = np.zeros((len(tiles), 128, S), np.float32)
    we = np.zeros((EDGE_VALID, S), np.float32)
    for j, t in enumerate(taps):
        t = int(t)
        if t // 128 == EDGE_TILE:
            we[t % 128, j] = 1.0
        else:
            w[pos[t // 128], t % 128, j] = 1.0
    return np.asarray(tiles, np.int32), w, we


_TILES, _W, _WE = _build_plan()
NT = len(_TILES)


def _body(x_ref, w_ref, we_ref, o_ref):
    acc = jnp.dot(
        x_ref[:, EDGE_TILE * 128:T],
        we_ref[...],
        preferred_element_type=jnp.float32,
    )
    for k, c in enumerate(_TILES):
        c = int(c)
        acc += jnp.dot(
            x_ref[:, c * 128:(c + 1) * 128],
            w_ref[k],
            preferred_element_type=jnp.float32,
        )
    o_ref[...] = acc


_gather = pl.pallas_call(
    _body,
    grid=(NBLK,),
    in_specs=[
        pl.BlockSpec((BR, T), lambda i: (i, 0)),
        pl.BlockSpec((NT, 128, S), lambda i: (0, 0, 0)),
        pl.BlockSpec((EDGE_VALID, S), lambda i: (0, 0)),
    ],
    out_specs=pl.BlockSpec((BR, S), lambda i: (i, 0)),
    out_shape=jax.ShapeDtypeStruct((B, S), jnp.float32),
    compiler_params=pltpu.CompilerParams(
        dimension_semantics=("arbitrary",),
    ),
)


@jax.jit
def kernel(x):
    x2d = jnp.squeeze(x, axis=1)
    return _gather(x2d, jnp.asarray(_W), jnp.asarray(_WE))


# speedup vs baseline: 1.1362x; 1.1362x over previous
import numpy as np
import jax
import jax.numpy as jnp
from jax.experimental import pallas as pl
from jax.experimental.pallas import tpu as pltpu

B, T, S = 4096, 10000, 128
BR = 512
NBLK = B // BR
EDGE_TILE = T // 128
EDGE_VALID = T - EDGE_TILE * 128


def _subsample_taps():
    max_logspace = np.log10(T - 1)
    idx = np.round(np.logspace(0.0, max_logspace, S, endpoint=True), 1).astype(int)
    idx[0] = 0
    return idx.astype(np.int32)


def _build_plan():
    taps = _subsample_taps()
    tiles = sorted(set(int(t) // 128 for t in taps if t // 128 != EDGE_TILE))
    pos = {c: k for k, c in enumerate(tiles)}
    w = np.zeros((len(tiles) + 1, 128, S), np.float32)
    for j, t in enumerate(taps):
        t = int(t)
        k = len(tiles) if t // 128 == EDGE_TILE else pos[t // 128]
        w[k, t % 128, j] = 1.0
    return tiles, w


_TILES, _W = _build_plan()
NT = len(_TILES)


def _body(*refs):
    w_ref = refs[NT + 1]
    o_ref = refs[NT + 2]
    lanes = jax.lax.broadcasted_iota(jnp.int32, (1, 128), 1)
    xe = jnp.where(lanes < EDGE_VALID, refs[NT][...], 0.0)
    acc = jnp.dot(xe, w_ref[NT], preferred_element_type=jnp.float32)
    for k in range(NT):
        acc += jnp.dot(
            refs[k][...], w_ref[k], preferred_element_type=jnp.float32
        )
    o_ref[...] = acc


_gather = pl.pallas_call(
    _body,
    grid=(NBLK,),
    in_specs=(
        [pl.BlockSpec((BR, 128), lambda i, c=c: (i, c)) for c in _TILES]
        + [
            pl.BlockSpec((BR, 128), lambda i: (i, EDGE_TILE)),
            pl.BlockSpec((NT + 1, 128, S), lambda i: (0, 0, 0)),
        ]
    ),
    out_specs=pl.BlockSpec((BR, S), lambda i: (i, 0)),
    out_shape=jax.ShapeDtypeStruct((B, S), jnp.float32),
    compiler_params=pltpu.CompilerParams(
        dimension_semantics=("arbitrary",),
    ),
)


@jax.jit
def kernel(x):
    x2d = jnp.squeeze(x, axis=1)
    return _gather(*([x2d] * (NT + 1)), jnp.asarray(_W))
